# Initial kernel scaffold; baseline (speedup 1.0000x reference)
import jax
import jax.numpy as jnp
from jax.experimental import pallas as pl


def kernel(x, edgeIndex, W0, b0, W1, b1, W2, b2, g0, be0, g1, be1, Wc1, bc1, Wc2, bc2):
    raise NotImplementedError("write your pallas kernel here")


# speedup vs baseline: 1.3135x; 1.3135x over previous
import functools

import jax
import jax.numpy as jnp
from jax.experimental import pallas as pl
from jax.experimental.pallas import tpu as pltpu

N = 10000
E = 320000
D_IN = 128
H = 256
C = 8


def _elu(x):
    return jnp.where(x > 0, x, jnp.exp(jnp.minimum(x, 0.0)) - 1.0)


def _bn_eval(x, gamma, beta):
    return (x / jnp.sqrt(1.0 + 1e-5)) * gamma + beta


def _gcn_conv(x, src, dst, W, b, dinv):
    norm = dinv[src] * dinv[dst]
    xw = x @ W
    msg = xw[src] * norm[:, None]
    out = jax.ops.segment_sum(msg, dst, num_segments=N)
    out = out + xw * (dinv * dinv)[:, None]
    return out + b


def _cls_body(h2_ref, wc1_ref, bc1_ref, wc2_ref, bc2_ref, out_ref):
    h2 = h2_ref[...]
    z = h2 @ wc1_ref[...] + bc1_ref[...]
    z = jnp.where(z > 0, z, jnp.exp(jnp.minimum(z, 0.0)) - 1.0)
    logits = jnp.dot(z, wc2_ref[...], preferred_element_type=jnp.float32)
    logits = logits + bc2_ref[...]
    m = jnp.max(logits, axis=-1, keepdims=True)
    lse = jnp.log(jnp.sum(jnp.exp(logits - m), axis=-1, keepdims=True)) + m
    out_ref[...] = logits - lse


def _classifier(h2, Wc1, bc1, Wc2, bc2):
    Wc2p = jnp.zeros((H // 2, 128), jnp.float32).at[:, :C].set(Wc2)
    bc2p = jnp.full((128,), -1e9, jnp.float32).at[:C].set(bc2)
    out = pl.pallas_call(
        _cls_body,
        out_shape=jax.ShapeDtypeStruct((N, 128), jnp.float32),
        grid=(10,),
        in_specs=[
            pl.BlockSpec((N // 10, H), lambda i: (i, 0)),
            pl.BlockSpec((H, H // 2), lambda i: (0, 0)),
            pl.BlockSpec((H // 2,), lambda i: (0,)),
            pl.BlockSpec((H // 2, 128), lambda i: (0, 0)),
            pl.BlockSpec((128,), lambda i: (0,)),
        ],
        out_specs=pl.BlockSpec((N // 10, 128), lambda i: (i, 0)),
    )(h2, Wc1, bc1, Wc2p, bc2p)
    return out[:, :C]


def kernel(x, edgeIndex, W0, b0, W1, b1, W2, b2, g0, be0, g1, be1, Wc1, bc1, Wc2, bc2):
    src = edgeIndex[0]
    dst = edgeIndex[1]
    deg = jax.ops.segment_sum(jnp.ones((E,), jnp.float32), dst, num_segments=N) + 1.0
    dinv = jax.lax.rsqrt(deg)
    h0 = _elu(_bn_eval(_gcn_conv(x, src, dst, W0, b0, dinv), g0, be0))
    h1 = _elu(_bn_eval(_gcn_conv(h0, src, dst, W1, b1, dinv) + h0, g1, be1))
    h2 = _gcn_conv(h1, src, dst, W2, b2, dinv) + h1
    logp = _classifier(h2, Wc1, bc1, Wc2, bc2)
    return logp, h2


# speedup vs baseline: 4.6771x; 3.5609x over previous
import functools

import jax
import jax.numpy as jnp
from jax import lax
from jax.experimental import pallas as pl
from jax.experimental.pallas import tpu as pltpu
from jax.experimental.pallas import tpu_sc as plsc

N = 10000
E = 320000
D_IN = 128
H = 256
C = 8

NP = 10240
NC = 2
NS = 16
TILES = NC * NS
PT = NP // TILES
CAP = 16384
K = 64
CH = 4000
_BN_C = 1.0 / (1.0 + 1e-5) ** 0.5


def _mesh():
    return plsc.VectorSubcoreMesh(core_axis_name="c", subcore_axis_name="s")


def _sc_prologue(edge):
    @functools.partial(
        pl.kernel,
        out_type=(
            jax.ShapeDtypeStruct((NP,), jnp.float32),
            jax.ShapeDtypeStruct((TILES, CAP), jnp.int32),
            jax.ShapeDtypeStruct((TILES, CAP), jnp.int32),
            jax.ShapeDtypeStruct((TILES, 16), jnp.int32),
        ),
        mesh=_mesh(),
        compiler_params=pltpu.CompilerParams(needs_layout_passes=False),
        scratch_types=[
            pltpu.VMEM((CH,), jnp.int32),
            pltpu.VMEM((CH,), jnp.int32),
            pltpu.VMEM((CAP,), jnp.int32),
            pltpu.VMEM((CAP,), jnp.int32),
            pltpu.VMEM((PT,), jnp.float32),
            pltpu.VMEM((16,), jnp.int32),
        ],
    )
    def prologue(edge_hbm, deg_hbm, srcl_hbm, dlocl_hbm, cnt_hbm,
                 srcb, dstb, srcl, dlocl, degl, cntv):
        cid = lax.axis_index("c")
        sid = lax.axis_index("s")
        wid = sid * NC + cid
        lo = wid * PT

        zf = jnp.zeros((16,), jnp.float32)

        def zdeg(i, carry):
            degl[pl.ds(i * 16, 16)] = zf
            return carry

        lax.fori_loop(0, PT // 16, zdeg, 0)

        lanes = lax.iota(jnp.int32, 16)
        fdl = jnp.full((16,), PT, jnp.int32)

        def zl(i, carry):
            srcl[pl.ds(i * 16, 16)] = (i * 16 + lanes) & 8191
            dlocl[pl.ds(i * 16, 16)] = fdl
            return carry

        lax.fori_loop(0, CAP // 16, zl, 0)

        ones = jnp.ones((16,), jnp.float32)

        def chunk_body(ci, cnt):
            base = ci * CH
            pltpu.sync_copy(edge_hbm.at[pl.ds(base, CH)], srcb)
            pltpu.sync_copy(edge_hbm.at[pl.ds(E + base, CH)], dstb)

            def vec_body(j, cnt):
                sv = srcb[pl.ds(j * 16, 16)]
                dv = dstb[pl.ds(j * 16, 16)]
                dl = dv - lo
                m = (dl >= 0) & (dl < PT)
                mi = m.astype(jnp.int32)
                pos = cnt + plsc.cumsum(mi) - 1
                plsc.store_scatter(srcl, [pos], sv, mask=m)
                plsc.store_scatter(dlocl, [pos], dl, mask=m)
                plsc.addupdate_scatter(degl, [dl], ones, mask=m)
                return cnt + jnp.sum(mi)

            return lax.fori_loop(0, CH // 16, vec_body, cnt)

        cnt = lax.fori_loop(0, E // CH, chunk_body, jnp.int32(0))
        cntv[...] = jnp.where(lax.iota(jnp.int32, 16) == 0, cnt, 0)
        pltpu.sync_copy(degl, deg_hbm.at[pl.ds(lo, PT)])
        pltpu.sync_copy(srcl, srcl_hbm.at[wid])
        pltpu.sync_copy(dlocl, dlocl_hbm.at[wid])
        pltpu.sync_copy(cntv, cnt_hbm.at[wid])

    return prologue(edge.reshape(2 * E))


def _sc_layer(y, srcl, dlocl, cnts):
    @functools.partial(
        pl.kernel,
        out_type=jax.ShapeDtypeStruct((NP, H), jnp.float32),
        mesh=_mesh(),
        compiler_params=pltpu.CompilerParams(needs_layout_passes=False),
        scratch_types=[
            pltpu.VMEM((PT + 1, H), jnp.float32),
            pltpu.VMEM((K,), jnp.int32),
            pltpu.VMEM((K,), jnp.int32),
            pltpu.VMEM((K, H), jnp.float32),
            pltpu.VMEM((16,), jnp.int32),
        ],
    )
    def layer(y_hbm, srcl_hbm, dlocl_hbm, cnt_hbm, out_hbm,
              acc, sidx, didx, rows, cntv):
        cid = lax.axis_index("c")
        sid = lax.axis_index("s")
        wid = sid * NC + cid
        lo = wid * PT

        pltpu.sync_copy(cnt_hbm.at[wid], cntv)
        cnt = jnp.sum(cntv[...])
        nch = (cnt + (K - 1)) // K

        zf = jnp.zeros((16,), jnp.float32)

        def zacc(r, carry):
            for u in range(H // 16):
                acc[r, pl.ds(u * 16, 16)] = zf
            return carry

        lax.fori_loop(0, PT + 1, zacc, 0)

        def chunk(i, carry):
            pltpu.sync_copy(srcl_hbm.at[wid, pl.ds(i * K, K)], sidx)
            pltpu.sync_copy(dlocl_hbm.at[wid, pl.ds(i * K, K)], didx)
            pltpu.sync_copy(y_hbm.at[sidx], rows)

            def group(g, c2):
                dlv = didx[pl.ds(g * 16, 16)]
                for u in range(16):
                    dl = dlv[u]
                    e = g * 16 + u
                    for f in range(H // 16):
                        v = rows[e, pl.ds(f * 16, 16)]
                        plsc.addupdate(acc.at[dl, pl.ds(f * 16, 16)], v)
                return c2

            lax.fori_loop(0, K // 16, group, 0)
            return carry

        lax.fori_loop(0, nch, chunk, 0)
        pltpu.sync_copy(acc.at[pl.ds(0, PT)], out_hbm.at[pl.ds(lo, PT)])

    return layer(y, srcl, dlocl, cnts)


def _k1_body(x_ref, w0_ref, deg_ref, y0_ref, dinv_ref):
    dinv = lax.rsqrt(deg_ref[...] + 1.0)
    xw = jnp.dot(x_ref[...], w0_ref[...], preferred_element_type=jnp.float32)
    y0_ref[...] = xw * dinv
    dinv_ref[...] = dinv


def _k1(x_pad, W0, deg):
    R = NP // 8
    return pl.pallas_call(
        _k1_body,
        out_shape=(
            jax.ShapeDtypeStruct((NP, H), jnp.float32),
            jax.ShapeDtypeStruct((NP, 1), jnp.float32),
        ),
        grid=(8,),
        in_specs=[
            pl.BlockSpec((R, D_IN), lambda i: (i, 0)),
            pl.BlockSpec((D_IN, H), lambda i: (0, 0)),
            pl.BlockSpec((R, 1), lambda i: (i, 0)),
        ],
        out_specs=(
            pl.BlockSpec((R, H), lambda i: (i, 0)),
            pl.BlockSpec((R, 1), lambda i: (i, 0)),
        ),
    )(x_pad, W0, deg.reshape(NP, 1))


def _elu(x):
    return jnp.where(x > 0, x, jnp.exp(jnp.minimum(x, 0.0)) - 1.0)


def _mid_body(s_ref, y_ref, res_ref, dinv_ref, w_ref, b_ref, g_ref, be_ref,
              h_ref, ynext_ref, *, residual):
    dinv = dinv_ref[...]
    conv = dinv * (s_ref[...] + y_ref[...]) + b_ref[...]
    if residual:
        conv = conv + res_ref[...]
    h = _elu(_BN_C * g_ref[...] * conv + be_ref[...])
    h_ref[...] = h
    ynext_ref[...] = jnp.dot(h, w_ref[...], preferred_element_type=jnp.float32) * dinv


def _k_mid(S, y, resid, dinv, Wn, b, g, be, residual):
    R = NP // 8
    return pl.pallas_call(
        functools.partial(_mid_body, residual=residual),
        out_shape=(
            jax.ShapeDtypeStruct((NP, H), jnp.float32),
            jax.ShapeDtypeStruct((NP, H), jnp.float32),
        ),
        grid=(8,),
        in_specs=[
            pl.BlockSpec((R, H), lambda i: (i, 0)),
            pl.BlockSpec((R, H), lambda i: (i, 0)),
            pl.BlockSpec((R, H), lambda i: (i, 0)),
            pl.BlockSpec((R, 1), lambda i: (i, 0)),
            pl.BlockSpec((H, H), lambda i: (0, 0)),
            pl.BlockSpec((H,), lambda i: (0,)),
            pl.BlockSpec((H,), lambda i: (0,)),
            pl.BlockSpec((H,), lambda i: (0,)),
        ],
        out_specs=(
            pl.BlockSpec((R, H), lambda i: (i, 0)),
            pl.BlockSpec((R, H), lambda i: (i, 0)),
        ),
    )(S, y, resid, dinv, Wn, b, g, be)


def _k4_body(s_ref, y_ref, res_ref, dinv_ref, b_ref, wc1_ref, bc1_ref,
             wc2_ref, bc2_ref, logp_ref, h2_ref):
    dinv = dinv_ref[...]
    h2 = dinv * (s_ref[...] + y_ref[...]) + b_ref[...] + res_ref[...]
    h2_ref[...] = h2
    z = _elu(jnp.dot(h2, wc1_ref[...], preferred_element_type=jnp.float32)
             + bc1_ref[...])
    logits = jnp.dot(z, wc2_ref[...], preferred_element_type=jnp.float32)
    logits = logits + bc2_ref[...]
    m = jnp.max(logits, axis=-1, keepdims=True)
    lse = jnp.log(jnp.sum(jnp.exp(logits - m), axis=-1, keepdims=True)) + m
    logp_ref[...] = logits - lse


def _k4(S2, y2, h1, dinv, b2, Wc1, bc1, Wc2p, bc2p):
    R = NP // 8
    return pl.pallas_call(
        _k4_body,
        out_shape=(
            jax.ShapeDtypeStruct((NP, 128), jnp.float32),
            jax.ShapeDtypeStruct((NP, H), jnp.float32),
        ),
        grid=(8,),
        in_specs=[
            pl.BlockSpec((R, H), lambda i: (i, 0)),
            pl.BlockSpec((R, H), lambda i: (i, 0)),
            pl.BlockSpec((R, H), lambda i: (i, 0)),
            pl.BlockSpec((R, 1), lambda i: (i, 0)),
            pl.BlockSpec((H,), lambda i: (0,)),
            pl.BlockSpec((H, H // 2), lambda i: (0, 0)),
            pl.BlockSpec((H // 2,), lambda i: (0,)),
            pl.BlockSpec((H // 2, 128), lambda i: (0, 0)),
            pl.BlockSpec((128,), lambda i: (0,)),
        ],
        out_specs=(
            pl.BlockSpec((R, 128), lambda i: (i, 0)),
            pl.BlockSpec((R, H), lambda i: (i, 0)),
        ),
    )(S2, y2, h1, dinv, b2, Wc1, bc1, Wc2p, bc2p)


def kernel(x, edgeIndex, W0, b0, W1, b1, W2, b2, g0, be0, g1, be1, Wc1, bc1, Wc2, bc2):
    x_pad = jnp.pad(x, ((0, NP - N), (0, 0)))
    Wc2p = jnp.zeros((H // 2, 128), jnp.float32).at[:, :C].set(Wc2)
    bc2p = jnp.full((128,), -1e9, jnp.float32).at[:C].set(bc2)

    deg, srcl, dlocl, cnts = _sc_prologue(edgeIndex)
    y0, dinv = _k1(x_pad, W0, deg)
    S0 = _sc_layer(y0, srcl, dlocl, cnts)
    h0, y1 = _k_mid(S0, y0, y0, dinv, W1, b0, g0, be0, residual=False)
    S1 = _sc_layer(y1, srcl, dlocl, cnts)
    h1, y2 = _k_mid(S1, y1, h0, dinv, W2, b1, g1, be1, residual=True)
    S2 = _sc_layer(y2, srcl, dlocl, cnts)
    logp, h2 = _k4(S2, y2, h1, dinv, b2, Wc1, bc1, Wc2p, bc2p)
    return logp[:N, :C], h2[:N]


# speedup vs baseline: 5.0706x; 1.0841x over previous
import functools

import jax
import jax.numpy as jnp
from jax import lax
from jax.experimental import pallas as pl
from jax.experimental.pallas import tpu as pltpu
from jax.experimental.pallas import tpu_sc as plsc

N = 10000
E = 320000
D_IN = 128
H = 256
C = 8

NP = 10240
NC = 2
NS = 16
TILES = NC * NS
PT = NP // TILES
CAP = 16384
K = 32
CH = 4000
_BN_C = 1.0 / (1.0 + 1e-5) ** 0.5


def _mesh():
    return plsc.VectorSubcoreMesh(core_axis_name="c", subcore_axis_name="s")


def _sc_prologue(edge):
    @functools.partial(
        pl.kernel,
        out_type=(
            jax.ShapeDtypeStruct((NP,), jnp.float32),
            jax.ShapeDtypeStruct((TILES, CAP), jnp.int32),
            jax.ShapeDtypeStruct((TILES, 16), jnp.int32),
        ),
        mesh=_mesh(),
        compiler_params=pltpu.CompilerParams(needs_layout_passes=False),
        scratch_types=[
            pltpu.VMEM((CH,), jnp.int32),
            pltpu.VMEM((CH,), jnp.int32),
            pltpu.VMEM((CAP,), jnp.int32),
            pltpu.VMEM((PT,), jnp.float32),
            pltpu.VMEM((16,), jnp.int32),
        ],
    )
    def prologue(edge_hbm, deg_hbm, pkl_hbm, cnt_hbm,
                 srcb, dstb, pkl, degl, cntv):
        cid = lax.axis_index("c")
        sid = lax.axis_index("s")
        wid = sid * NC + cid
        lo = wid * PT

        zf = jnp.zeros((16,), jnp.float32)

        def zdeg(i, carry):
            degl[pl.ds(i * 16, 16)] = zf
            return carry

        lax.fori_loop(0, PT // 16, zdeg, 0)

        lanes = lax.iota(jnp.int32, 16)

        def zl(i, carry):
            pkl[pl.ds(i * 16, 16)] = PT * 16384 + ((i * 16 + lanes) & 8191)
            return carry

        lax.fori_loop(0, CAP // 16, zl, 0)

        ones = jnp.ones((16,), jnp.float32)

        def chunk_body(ci, cnt):
            base = ci * CH
            pltpu.sync_copy(edge_hbm.at[pl.ds(base, CH)], srcb)
            pltpu.sync_copy(edge_hbm.at[pl.ds(E + base, CH)], dstb)

            def vec_body(j, cnt):
                sv = srcb[pl.ds(j * 16, 16)]
                dv = dstb[pl.ds(j * 16, 16)]
                dl = dv - lo
                m = (dl >= 0) & (dl < PT)
                mi = m.astype(jnp.int32)
                pos = cnt + plsc.cumsum(mi) - 1
                plsc.store_scatter(pkl, [pos], dl * 16384 + sv, mask=m)
                plsc.addupdate_scatter(degl, [dl], ones, mask=m)
                return pos[15] + 1

            return lax.fori_loop(0, CH // 16, vec_body, cnt)

        cnt = lax.fori_loop(0, E // CH, chunk_body, jnp.int32(0))
        cntv[...] = jnp.where(lax.iota(jnp.int32, 16) == 0, cnt, 0)
        pltpu.sync_copy(degl, deg_hbm.at[pl.ds(lo, PT)])
        pltpu.sync_copy(pkl, pkl_hbm.at[wid])
        pltpu.sync_copy(cntv, cnt_hbm.at[wid])

    return prologue(edge.reshape(2 * E))


def _sc_layer(y, pkl, cnts):
    @functools.partial(
        pl.kernel,
        out_type=jax.ShapeDtypeStruct((NP, H), jnp.float32),
        mesh=_mesh(),
        compiler_params=pltpu.CompilerParams(needs_layout_passes=False),
        scratch_types=[
            pltpu.VMEM((PT + 1, H), jnp.float32),
            pltpu.VMEM((CAP,), jnp.int32),
            pltpu.VMEM((2, K), jnp.int32),
            pltpu.VMEM((2, K, H), jnp.float32),
            pltpu.VMEM((16,), jnp.int32),
            pltpu.SemaphoreType.DMA((2,)),
        ],
    )
    def layer(y_hbm, pkl_hbm, cnt_hbm, out_hbm,
              acc, pkl, sidx, rows, cntv, sems):
        cid = lax.axis_index("c")
        sid = lax.axis_index("s")
        wid = sid * NC + cid
        lo = wid * PT

        pltpu.sync_copy(cnt_hbm.at[wid], cntv)
        pltpu.sync_copy(pkl_hbm.at[wid], pkl)
        cnt = jnp.sum(cntv[...])
        nch = (cnt + (K - 1)) // K

        zf = jnp.zeros((16,), jnp.float32)

        def zacc(r, carry):
            for u in range(H // 16):
                acc[r, pl.ds(u * 16, 16)] = zf
            return carry

        lax.fori_loop(0, PT + 1, zacc, 0)

        def issue(i, b):
            for g in range(K // 16):
                pkv = pkl[pl.ds(i * K + g * 16, 16)]
                sidx[b, pl.ds(g * 16, 16)] = pkv & 16383
            pltpu.async_copy(y_hbm.at[sidx.at[b]], rows.at[b], sems.at[b])

        @pl.when(nch > 0)
        def _():
            issue(0, 0)

        def chunk(i, carry):
            b = lax.rem(i, 2)

            @pl.when(i + 1 < nch)
            def _():
                issue(i + 1, 1 - b)

            pltpu.make_async_copy(
                y_hbm.at[sidx.at[b]], rows.at[b], sems.at[b]
            ).wait()
            for g in range(K // 16):
                pkv = pkl[pl.ds(i * K + g * 16, 16)]
                dlv = lax.shift_right_logical(pkv, 14)
                for u in range(16):
                    dl = dlv[u]
                    e = g * 16 + u
                    for f in range(H // 16):
                        v = rows[b, e, pl.ds(f * 16, 16)]
                        plsc.addupdate(acc.at[dl, pl.ds(f * 16, 16)], v)

            return carry

        lax.fori_loop(0, nch, chunk, 0)
        pltpu.sync_copy(acc.at[pl.ds(0, PT)], out_hbm.at[pl.ds(lo, PT)])

    return layer(y, pkl, cnts)


def _k1_body(x_ref, w0_ref, deg_ref, y0_ref, dinv_ref):
    dinv = lax.rsqrt(deg_ref[...] + 1.0)
    xw = jnp.dot(x_ref[...], w0_ref[...], preferred_element_type=jnp.float32)
    y0_ref[...] = xw * dinv
    dinv_ref[...] = dinv


def _k1(x_pad, W0, deg):
    R = NP // 8
    return pl.pallas_call(
        _k1_body,
        out_shape=(
            jax.ShapeDtypeStruct((NP, H), jnp.float32),
            jax.ShapeDtypeStruct((NP, 1), jnp.float32),
        ),
        grid=(8,),
        in_specs=[
            pl.BlockSpec((R, D_IN), lambda i: (i, 0)),
            pl.BlockSpec((D_IN, H), lambda i: (0, 0)),
            pl.BlockSpec((R, 1), lambda i: (i, 0)),
        ],
        out_specs=(
            pl.BlockSpec((R, H), lambda i: (i, 0)),
            pl.BlockSpec((R, 1), lambda i: (i, 0)),
        ),
    )(x_pad, W0, deg.reshape(NP, 1))


def _elu(x):
    return jnp.where(x > 0, x, jnp.exp(jnp.minimum(x, 0.0)) - 1.0)


def _mid_body(s_ref, y_ref, res_ref, dinv_ref, w_ref, b_ref, g_ref, be_ref,
              h_ref, ynext_ref, *, residual):
    dinv = dinv_ref[...]
    conv = dinv * (s_ref[...] + y_ref[...]) + b_ref[...]
    if residual:
        conv = conv + res_ref[...]
    h = _elu(_BN_C * g_ref[...] * conv + be_ref[...])
    h_ref[...] = h
    ynext_ref[...] = jnp.dot(h, w_ref[...], preferred_element_type=jnp.float32) * dinv


def _k_mid(S, y, resid, dinv, Wn, b, g, be, residual):
    R = NP // 8
    return pl.pallas_call(
        functools.partial(_mid_body, residual=residual),
        out_shape=(
            jax.ShapeDtypeStruct((NP, H), jnp.float32),
            jax.ShapeDtypeStruct((NP, H), jnp.float32),
        ),
        grid=(8,),
        in_specs=[
            pl.BlockSpec((R, H), lambda i: (i, 0)),
            pl.BlockSpec((R, H), lambda i: (i, 0)),
            pl.BlockSpec((R, H), lambda i: (i, 0)),
            pl.BlockSpec((R, 1), lambda i: (i, 0)),
            pl.BlockSpec((H, H), lambda i: (0, 0)),
            pl.BlockSpec((H,), lambda i: (0,)),
            pl.BlockSpec((H,), lambda i: (0,)),
            pl.BlockSpec((H,), lambda i: (0,)),
        ],
        out_specs=(
            pl.BlockSpec((R, H), lambda i: (i, 0)),
            pl.BlockSpec((R, H), lambda i: (i, 0)),
        ),
    )(S, y, resid, dinv, Wn, b, g, be)


def _k4_body(s_ref, y_ref, res_ref, dinv_ref, b_ref, wc1_ref, bc1_ref,
             wc2_ref, bc2_ref, logp_ref, h2_ref):
    dinv = dinv_ref[...]
    h2 = dinv * (s_ref[...] + y_ref[...]) + b_ref[...] + res_ref[...]
    h2_ref[...] = h2
    z = _elu(jnp.dot(h2, wc1_ref[...], preferred_element_type=jnp.float32)
             + bc1_ref[...])
    logits = jnp.dot(z, wc2_ref[...], preferred_element_type=jnp.float32)
    logits = logits + bc2_ref[...]
    m = jnp.max(logits, axis=-1, keepdims=True)
    lse = jnp.log(jnp.sum(jnp.exp(logits - m), axis=-1, keepdims=True)) + m
    logp_ref[...] = logits - lse


def _k4(S2, y2, h1, dinv, b2, Wc1, bc1, Wc2p, bc2p):
    R = NP // 8
    return pl.pallas_call(
        _k4_body,
        out_shape=(
            jax.ShapeDtypeStruct((NP, 128), jnp.float32),
            jax.ShapeDtypeStruct((NP, H), jnp.float32),
        ),
        grid=(8,),
        in_specs=[
            pl.BlockSpec((R, H), lambda i: (i, 0)),
            pl.BlockSpec((R, H), lambda i: (i, 0)),
            pl.BlockSpec((R, H), lambda i: (i, 0)),
            pl.BlockSpec((R, 1), lambda i: (i, 0)),
            pl.BlockSpec((H,), lambda i: (0,)),
            pl.BlockSpec((H, H // 2), lambda i: (0, 0)),
            pl.BlockSpec((H // 2,), lambda i: (0,)),
            pl.BlockSpec((H // 2, 128), lambda i: (0, 0)),
            pl.BlockSpec((128,), lambda i: (0,)),
        ],
        out_specs=(
            pl.BlockSpec((R, 128), lambda i: (i, 0)),
            pl.BlockSpec((R, H), lambda i: (i, 0)),
        ),
    )(S2, y2, h1, dinv, b2, Wc1, bc1, Wc2p, bc2p)


def kernel(x, edgeIndex, W0, b0, W1, b1, W2, b2, g0, be0, g1, be1, Wc1, bc1, Wc2, bc2):
    x_pad = jnp.pad(x, ((0, NP - N), (0, 0)))
    Wc2p = jnp.zeros((H // 2, 128), jnp.float32).at[:, :C].set(Wc2)
    bc2p = jnp.full((128,), -1e9, jnp.float32).at[:C].set(bc2)

    deg, pkl, cnts = _sc_prologue(edgeIndex)
    y0, dinv = _k1(x_pad, W0, deg)
    S0 = _sc_layer(y0, pkl, cnts)
    h0, y1 = _k_mid(S0, y0, y0, dinv, W1, b0, g0, be0, residual=False)
    S1 = _sc_layer(y1, pkl, cnts)
    h1, y2 = _k_mid(S1, y1, h0, dinv, W2, b1, g1, be1, residual=True)
    S2 = _sc_layer(y2, pkl, cnts)
    logp, h2 = _k4(S2, y2, h1, dinv, b2, Wc1, bc1, Wc2p, bc2p)
    return logp[:N, :C], h2[:N]
